# baseline (speedup 1.0000x reference)
import numpy as np
import jax
import jax.numpy as jnp
from jax import lax
from jax.experimental import pallas as pl
from jax.experimental.pallas import tpu as pltpu

_B, _F, _D = 4096, 100, 64
_C = _F * _D
_ROWS_PER_BLOCK = 64

_U32 = np.uint32


def _np_threefry2x32(k1, k2, x0, x1):
    u = lambda v: np.array(v, dtype=_U32)
    rotl = lambda x, d: (x << u(d)) | (x >> u(32 - d))
    ks = [u(k1), u(k2), u(k1) ^ u(k2) ^ u(0x1BD11BDA)]
    rot = [(13, 15, 26, 6), (17, 29, 16, 24)]
    x0, x1 = u(x0) + ks[0], u(x1) + ks[1]
    for i in range(5):
        for r in rot[i % 2]:
            x0 = x0 + x1
            x1 = rotl(x1, r) ^ x0
        x0 = x0 + ks[(i + 1) % 3]
        x1 = x1 + ks[(i + 2) % 3] + u(i + 1)
    return x0, x1


with np.errstate(over="ignore"):
    _KM = _np_threefry2x32(0, 42, 0, 0)
    _KN = _np_threefry2x32(0, 42, 0, 1)

_LO = np.nextafter(np.float32(-1.0), np.float32(0.0), dtype=np.float32)
_HILO = np.float32(np.float32(1.0) - _LO)
_SCALE = np.float32(np.float32(np.sqrt(2.0)) * np.float32(0.05))
_P_CENTER = [np.float32(c) * _SCALE for c in (
    2.81022636e-08, 3.43273939e-07, -3.5233877e-06, -4.39150654e-06,
    0.00021858087, -0.00125372503, -0.00417768164, 0.246640727, 1.50140941)]
_P_TAIL = [np.float32(c) * _SCALE for c in (
    -0.000200214257, 0.000100950558, 0.00134934322, -0.00367342844,
    0.00573950773, -0.0076224613, 0.00943887047, 1.00167406, 2.83297682)]

_EXPAND = np.kron(np.eye(_F, dtype=np.float32),
                  np.ones((1, _D), dtype=np.float32))


def _tf_rounds(k0, k1, x0, x1):
    ks = [_U32(k0), _U32(k1), _U32(k0) ^ _U32(k1) ^ _U32(0x1BD11BDA)]
    rot = [(13, 15, 26, 6), (17, 29, 16, 24)]
    x0 = x0 + ks[0]
    x1 = x1 + ks[1]
    for i in range(5):
        for r in rot[i % 2]:
            x0 = x0 + x1
            x1 = ((x1 << _U32(r)) | (x1 >> _U32(32 - r))) ^ x0
        x0 = x0 + ks[(i + 1) % 3]
        x1 = x1 + (ks[(i + 2) % 3] + _U32(i + 1))
    return x0, x1


def _bits(key, flat_idx_u32):
    o0, o1 = _tf_rounds(key[0], key[1], jnp.zeros_like(flat_idx_u32), flat_idx_u32)
    return o0 ^ o1


def _kern(x_ref, m_ref, e_ref, o_ref):
    rb = x_ref.shape[0]
    row0 = pl.program_id(0) * rb

    r_i = lax.broadcasted_iota(jnp.int32, (rb, _F), 0)
    f_i = lax.broadcasted_iota(jnp.int32, (rb, _F), 1)
    midx = ((row0 + r_i) * _F + f_i).astype(jnp.uint32)
    mbits = _bits(_KM, midx)
    aug = jnp.where((mbits >> _U32(31)) == _U32(0),
                    jnp.float32(1.0), jnp.float32(0.0)) * m_ref[...]
    e = lax.dot_general(aug, e_ref[...], (((1,), (0,)), ((), ())),
                        preferred_element_type=jnp.float32)

    r_j = lax.broadcasted_iota(jnp.int32, (rb, _C), 0)
    c_j = lax.broadcasted_iota(jnp.int32, (rb, _C), 1)
    nidx = ((row0 + r_j) * _C + c_j).astype(jnp.uint32)
    nbits = _bits(_KN, nidx)
    fb = (nbits >> _U32(9)) | _U32(0x3F800000)
    f01 = lax.bitcast_convert_type(fb, jnp.float32) - jnp.float32(1.0)
    u = jnp.maximum(f01 * _HILO + _LO, _LO)
    w = -jnp.log(jnp.float32(1.0) - u * u)
    tail = w >= jnp.float32(5.0)
    ww = jnp.where(tail, jnp.sqrt(w) - jnp.float32(3.0), w - jnp.float32(2.5))
    p = jnp.where(tail, _P_TAIL[0], _P_CENTER[0])
    for cc, ct in zip(_P_CENTER[1:], _P_TAIL[1:]):
        p = p * ww + jnp.where(tail, ct, cc)
    pert = p * u

    o_ref[...] = x_ref[...] + pert * e


def kernel(input_features, attention_mask):
    x = input_features.reshape(_B, _C)
    attn = attention_mask.astype(jnp.float32)
    nblk = _B // _ROWS_PER_BLOCK
    out = pl.pallas_call(
        _kern,
        grid=(nblk,),
        in_specs=[
            pl.BlockSpec((_ROWS_PER_BLOCK, _C), lambda i: (i, 0)),
            pl.BlockSpec((_ROWS_PER_BLOCK, _F), lambda i: (i, 0)),
            pl.BlockSpec((_F, _C), lambda i: (0, 0)),
        ],
        out_specs=pl.BlockSpec((_ROWS_PER_BLOCK, _C), lambda i: (i, 0)),
        out_shape=jax.ShapeDtypeStruct((_B, _C), jnp.float32),
    )(x, attn, _EXPAND)
    return out.reshape(_B, _F, _D)
